# baseline (speedup 1.0000x reference)
import functools

import jax
import jax.numpy as jnp
from jax.experimental import pallas as pl
from jax.experimental.pallas import tpu as pltpu

_W = 512


def _round_up(x, m):
    return (x + m - 1) // m * m


def _pick_tile(npad, max_tile):
    cap = min(max_tile, npad if npad < 256 else npad // 2)
    tm = 128
    cand = 128
    while cand <= cap:
        if npad % cand == 0:
            tm = cand
        cand += 128
    return tm


def _build_kernel(starts_ref, key_ref, x_ref, w1_ref, a_ref, dinv_ref,
                  p1_ref, acc_ref, deg_ref, *, tb, lb, n):
    nt = pl.num_programs(1)
    gi = pl.program_id(0)
    gk = pl.program_id(1)
    t = gi * nt + gk
    start = starts_ref[t]
    end = starts_ref[t + 1]
    base0 = (start // 128) * 128
    nw = (end - base0 + (_W - 1)) // _W

    iota = jax.lax.broadcasted_iota(jnp.int32, (tb, _W), 0).astype(jnp.int16)

    def window(w):
        base = pl.multiple_of(base0 + w * _W, 128)
        kwx = key_ref[:, pl.ds(base, _W + 128)]
        kw = kwx[:, 128:]
        kwprev = kwx[:, 127:127 + _W]
        pos = base + jax.lax.broadcasted_iota(jnp.int32, (1, _W), 1)
        valid = (pos >= start) & (pos < end) & (kw != kwprev)
        lr = jnp.where(valid, (kw >> lb) & (tb - 1), -1).astype(jnp.int16)
        lc = (kw & (tb - 1)).astype(jnp.int16)
        r_oh = (lr == iota).astype(jnp.int8)
        c_oh = (lc == iota).astype(jnp.int8)
        return jax.lax.dot_general(
            r_oh, c_oh, (((1,), (1,)), ((), ())),
            preferred_element_type=jnp.int32)

    acc_ref[...] = window(0)

    def body(w, carry):
        acc_ref[...] += window(w)
        return carry

    jax.lax.fori_loop(1, nw, body, 0, unroll=False)

    a_tile = acc_ref[...]
    a_ref[...] = a_tile.astype(jnp.int8)

    deg_part = jnp.sum(a_tile, axis=1, keepdims=True)

    @pl.when(gk == 0)
    def _():
        deg_ref[...] = deg_part

    @pl.when(gk > 0)
    def _():
        deg_ref[...] += deg_part

    @pl.when(gk == nt - 1)
    def _():
        row = gi * tb + jax.lax.broadcasted_iota(jnp.int32, (tb, 1), 0)
        deg = (deg_ref[...] + jnp.where(row < n, 1, 0)).astype(jnp.float32)
        dinv = jnp.where(deg > 0, jax.lax.rsqrt(jnp.maximum(deg, 1.0)), 0.0)
        dinv_ref[...] = dinv
        p1 = jnp.dot(x_ref[...], w1_ref[...],
                     preferred_element_type=jnp.float32)
        p1_ref[...] = (dinv * p1).astype(p1_ref.dtype)


def _agg_proj_kernel(a_ref, h_ref, dinv_ref, b_ref, w2_ref, o_ref, acc_ref):
    i = pl.program_id(0)
    k = pl.program_id(1)
    tm = a_ref.shape[1]

    @pl.when(k == 0)
    def _():
        acc_ref[...] = jnp.zeros_like(acc_ref)

    a = a_ref[...].astype(jnp.bfloat16)
    start = pl.multiple_of(k * tm, tm)
    h_blk = h_ref[pl.ds(start, tm), :]
    acc_ref[...] += jnp.dot(a, h_blk, preferred_element_type=jnp.float32)

    @pl.when(k == i)
    def _():
        acc_ref[...] += h_blk.astype(jnp.float32)

    @pl.when(k == pl.num_programs(1) - 1)
    def _():
        o1 = jnp.maximum(dinv_ref[...] * acc_ref[...] + b_ref[...], 0.0)
        p2 = jnp.dot(o1.astype(jnp.bfloat16), w2_ref[...],
                     preferred_element_type=jnp.float32)
        o_ref[...] = (dinv_ref[...] * p2).astype(o_ref.dtype)


def _agg_final_kernel(a_ref, h_ref, dinv_ref, b_ref, o_ref):
    i = pl.program_id(0)
    k = pl.program_id(1)
    tm = a_ref.shape[1]

    @pl.when(k == 0)
    def _():
        o_ref[...] = jnp.zeros_like(o_ref)

    a = a_ref[...].astype(jnp.bfloat16)
    start = pl.multiple_of(k * tm, tm)
    h_blk = h_ref[pl.ds(start, tm), :]
    o_ref[...] += jnp.dot(a, h_blk, preferred_element_type=jnp.float32)

    @pl.when(k == i)
    def _():
        o_ref[...] += h_blk.astype(jnp.float32)

    @pl.when(k == pl.num_programs(1) - 1)
    def _():
        o_ref[...] = jnp.maximum(dinv_ref[...] * o_ref[...] + b_ref[...], 0.0)


def kernel(x, edge_index, w1, b1, w2, b2):
    n, f_in = x.shape
    hidden = w1.shape[1]

    npad = _round_up(n, 128)
    f_pad = _round_up(f_in, 128)
    h_pad = _round_up(hidden, 128)
    tm = _pick_tile(npad, 512)
    n_row = npad // tm

    tb = next(t for t in (512, 256, 128) if npad % t == 0)
    lb = tb.bit_length() - 1
    nt = npad // tb

    src = edge_index[0].astype(jnp.int32)
    dst0 = edge_index[1].astype(jnp.int32)
    dst = jnp.where(src == dst0, npad, dst0)
    key = ((((dst >> lb) * nt + (src >> lb)) << (2 * lb))
           | ((dst & (tb - 1)) << lb) | (src & (tb - 1)))
    num_e = key.shape[0]
    e_pad = _round_up(num_e + _W, 128)
    sentinel = jnp.int32((nt * nt + nt + 1) << (2 * lb))
    skey = jax.lax.sort(jnp.pad(key, (0, e_pad - num_e),
                                constant_values=sentinel), is_stable=False)
    cuts = (jnp.arange(nt * nt + 1, dtype=jnp.int32) << (2 * lb))
    starts = jnp.searchsorted(skey, cuts, side="left",
                              method="compare_all").astype(jnp.int32)
    key2d = jnp.concatenate(
        [jnp.full((128,), -1, jnp.int32), skey]).reshape(1, e_pad + 128)

    x_p = jnp.zeros((npad, f_pad), jnp.bfloat16).at[:n, :f_in].set(
        x.astype(jnp.bfloat16))
    w1_p = jnp.zeros((f_pad, h_pad), jnp.bfloat16).at[:f_in, :hidden].set(
        w1.astype(jnp.bfloat16))
    b1_p = jnp.zeros((1, h_pad), jnp.float32).at[0, :hidden].set(b1)
    w2_p = jnp.zeros((h_pad, h_pad), jnp.bfloat16).at[:hidden, :hidden].set(
        w2.astype(jnp.bfloat16))
    b2_p = jnp.zeros((1, h_pad), jnp.float32).at[0, :hidden].set(b2)

    a_p, dinv_p, p1 = pl.pallas_call(
        functools.partial(_build_kernel, tb=tb, lb=lb, n=n),
        grid_spec=pltpu.PrefetchScalarGridSpec(
            num_scalar_prefetch=1,
            grid=(nt, nt),
            in_specs=[
                pl.BlockSpec((1, e_pad + 128), lambda i, k, s: (0, 0)),
                pl.BlockSpec((tb, f_pad), lambda i, k, s: (i, 0)),
                pl.BlockSpec((f_pad, h_pad), lambda i, k, s: (0, 0)),
            ],
            out_specs=[
                pl.BlockSpec((tb, tb), lambda i, k, s: (i, k)),
                pl.BlockSpec((tb, 1), lambda i, k, s: (i, 0)),
                pl.BlockSpec((tb, h_pad), lambda i, k, s: (i, 0)),
            ],
            scratch_shapes=[pltpu.VMEM((tb, tb), jnp.int32),
                            pltpu.VMEM((tb, 1), jnp.int32)],
        ),
        out_shape=[jax.ShapeDtypeStruct((npad, npad), jnp.int8),
                   jax.ShapeDtypeStruct((npad, 1), jnp.float32),
                   jax.ShapeDtypeStruct((npad, h_pad), jnp.bfloat16)],
        compiler_params=pltpu.CompilerParams(
            dimension_semantics=("parallel", "arbitrary")),
    )(starts, key2d, x_p, w1_p)

    vmem_limit = min(
        int(2 * tm * tm
            + 2 * npad * h_pad * 2
            + 2 * tm * 128 * 4
            + 2 * 8 * h_pad * 4
            + h_pad * h_pad * 2
            + tm * h_pad * 4
            + 2 * tm * h_pad * 4) * 2
        + (8 << 20), 96 << 20)

    agg_cost = pl.CostEstimate(
        flops=2 * npad * npad * h_pad,
        transcendentals=0,
        bytes_accessed=npad * npad + npad * h_pad * 2 + npad * h_pad * 4
                       + npad * 4,
    )

    p2 = pl.pallas_call(
        _agg_proj_kernel,
        out_shape=jax.ShapeDtypeStruct((npad, h_pad), jnp.bfloat16),
        grid=(n_row, n_row),
        in_specs=[
            pl.BlockSpec((tm, tm), lambda i, k: (i, k)),
            pl.BlockSpec((npad, h_pad), lambda i, k: (0, 0)),
            pl.BlockSpec((tm, 1), lambda i, k: (i, 0)),
            pl.BlockSpec((1, h_pad), lambda i, k: (0, 0)),
            pl.BlockSpec((h_pad, h_pad), lambda i, k: (0, 0)),
        ],
        out_specs=pl.BlockSpec((tm, h_pad), lambda i, k: (i, 0)),
        scratch_shapes=[pltpu.VMEM((tm, h_pad), jnp.float32)],
        compiler_params=pltpu.CompilerParams(
            dimension_semantics=("parallel", "arbitrary"),
            vmem_limit_bytes=vmem_limit,
        ),
        cost_estimate=agg_cost,
    )(a_p, p1, dinv_p, b1_p, w2_p)

    out = pl.pallas_call(
        _agg_final_kernel,
        out_shape=jax.ShapeDtypeStruct((npad, h_pad), jnp.float32),
        grid=(n_row, n_row),
        in_specs=[
            pl.BlockSpec((tm, tm), lambda i, k: (i, k)),
            pl.BlockSpec((npad, h_pad), lambda i, k: (0, 0)),
            pl.BlockSpec((tm, 1), lambda i, k: (i, 0)),
            pl.BlockSpec((1, h_pad), lambda i, k: (0, 0)),
        ],
        out_specs=pl.BlockSpec((tm, h_pad), lambda i, k: (i, 0)),
        compiler_params=pltpu.CompilerParams(
            dimension_semantics=("parallel", "arbitrary"),
            vmem_limit_bytes=vmem_limit,
        ),
        cost_estimate=agg_cost,
    )(a_p, p2, dinv_p, b2_p)

    return out[:n, :hidden]


# speedup vs baseline: 1.6472x; 1.6472x over previous
import functools

import jax
import jax.numpy as jnp
from jax.experimental import pallas as pl
from jax.experimental.pallas import tpu as pltpu

_W = 512


def _round_up(x, m):
    return (x + m - 1) // m * m


def _pick_tile(npad, max_tile):
    cap = min(max_tile, npad if npad < 256 else npad // 2)
    tm = 128
    cand = 128
    while cand <= cap:
        if npad % cand == 0:
            tm = cand
        cand += 128
    return tm


def _build_kernel(starts_ref, key_ref, x_ref, w1_ref, a_ref, dinv_ref,
                  p1_ref, acc_ref, deg_ref, *, tb, lb, n):
    nt = pl.num_programs(1)
    gi = pl.program_id(0)
    gk = pl.program_id(1)
    t = gi * nt + gk
    start = starts_ref[t]
    end = starts_ref[t + 1]
    base0 = (start // 128) * 128
    nw = (end - base0 + (_W - 1)) // _W

    iota = jax.lax.broadcasted_iota(jnp.int32, (tb, _W), 0).astype(jnp.int16)

    def window(w):
        base = pl.multiple_of(base0 + w * _W, 128)
        kwx = key_ref[:, pl.ds(base, _W + 128)]
        kw = kwx[:, 128:]
        kwprev = kwx[:, 127:127 + _W]
        pos = base + jax.lax.broadcasted_iota(jnp.int32, (1, _W), 1)
        valid = (pos >= start) & (pos < end) & (kw != kwprev)
        lr = jnp.where(valid, (kw >> lb) & (tb - 1), -1).astype(jnp.int16)
        lc = (kw & (tb - 1)).astype(jnp.int16)
        r_oh = (lr == iota).astype(jnp.int8)
        c_oh = (lc == iota).astype(jnp.int8)
        return jax.lax.dot_general(
            r_oh, c_oh, (((1,), (1,)), ((), ())),
            preferred_element_type=jnp.int32)

    acc_ref[...] = window(0)

    def body(w, carry):
        acc_ref[...] += window(w)
        return carry

    jax.lax.fori_loop(1, nw, body, 0, unroll=False)

    a_tile = acc_ref[...]
    a_ref[...] = a_tile.astype(jnp.int8)

    deg_part = jnp.sum(a_tile, axis=1, keepdims=True)

    @pl.when(gk == 0)
    def _():
        deg_ref[...] = deg_part

    @pl.when(gk > 0)
    def _():
        deg_ref[...] += deg_part

    @pl.when(gk == nt - 1)
    def _():
        row = gi * tb + jax.lax.broadcasted_iota(jnp.int32, (tb, 1), 0)
        deg = (deg_ref[...] + jnp.where(row < n, 1, 0)).astype(jnp.float32)
        dinv = jnp.where(deg > 0, jax.lax.rsqrt(jnp.maximum(deg, 1.0)), 0.0)
        dinv_ref[...] = dinv
        p1 = jnp.dot(x_ref[...], w1_ref[...],
                     preferred_element_type=jnp.float32)
        p1_ref[...] = (dinv * p1).astype(p1_ref.dtype)


def _agg_proj_kernel(a_ref, h_ref, dinv_ref, b_ref, w2_ref, o_ref, acc_ref):
    i = pl.program_id(0)
    k = pl.program_id(1)
    tm = a_ref.shape[1]

    @pl.when(k == 0)
    def _():
        acc_ref[...] = jnp.zeros_like(acc_ref)

    a = a_ref[...].astype(jnp.bfloat16)
    start = pl.multiple_of(k * tm, tm)
    h_blk = h_ref[pl.ds(start, tm), :]
    acc_ref[...] += jnp.dot(a, h_blk, preferred_element_type=jnp.float32)

    @pl.when(k == i)
    def _():
        acc_ref[...] += h_blk.astype(jnp.float32)

    @pl.when(k == pl.num_programs(1) - 1)
    def _():
        o1 = jnp.maximum(dinv_ref[...] * acc_ref[...] + b_ref[...], 0.0)
        p2 = jnp.dot(o1.astype(jnp.bfloat16), w2_ref[...],
                     preferred_element_type=jnp.float32)
        o_ref[...] = (dinv_ref[...] * p2).astype(o_ref.dtype)


def _agg_final_kernel(a_ref, h_ref, dinv_ref, b_ref, o_ref):
    i = pl.program_id(0)
    k = pl.program_id(1)
    tm = a_ref.shape[1]

    @pl.when(k == 0)
    def _():
        o_ref[...] = jnp.zeros_like(o_ref)

    a = a_ref[...].astype(jnp.bfloat16)
    start = pl.multiple_of(k * tm, tm)
    h_blk = h_ref[pl.ds(start, tm), :]
    o_ref[...] += jnp.dot(a, h_blk, preferred_element_type=jnp.float32)

    @pl.when(k == i)
    def _():
        o_ref[...] += h_blk.astype(jnp.float32)

    @pl.when(k == pl.num_programs(1) - 1)
    def _():
        o_ref[...] = jnp.maximum(dinv_ref[...] * o_ref[...] + b_ref[...], 0.0)


def kernel(x, edge_index, w1, b1, w2, b2):
    n, f_in = x.shape
    hidden = w1.shape[1]

    npad = _round_up(n, 128)
    f_pad = _round_up(f_in, 128)
    h_pad = _round_up(hidden, 128)
    tm = _pick_tile(npad, 2048)
    n_row = npad // tm

    tb = next(t for t in (512, 256, 128) if npad % t == 0)
    lb = tb.bit_length() - 1
    nt = npad // tb

    src = edge_index[0].astype(jnp.int32)
    dst0 = edge_index[1].astype(jnp.int32)
    dst = jnp.where(src == dst0, npad, dst0)
    key = ((((dst >> lb) * nt + (src >> lb)) << (2 * lb))
           | ((dst & (tb - 1)) << lb) | (src & (tb - 1)))
    num_e = key.shape[0]
    e_pad = _round_up(num_e + _W, 128)
    sentinel = jnp.int32((nt * nt + nt + 1) << (2 * lb))
    skey = jax.lax.sort(jnp.pad(key, (0, e_pad - num_e),
                                constant_values=sentinel), is_stable=False)
    cuts = (jnp.arange(nt * nt + 1, dtype=jnp.int32) << (2 * lb))
    starts = jnp.searchsorted(skey, cuts, side="left",
                              method="compare_all").astype(jnp.int32)
    key2d = jnp.concatenate(
        [jnp.full((128,), -1, jnp.int32), skey]).reshape(1, e_pad + 128)

    x_p = jnp.zeros((npad, f_pad), jnp.bfloat16).at[:n, :f_in].set(
        x.astype(jnp.bfloat16))
    w1_p = jnp.zeros((f_pad, h_pad), jnp.bfloat16).at[:f_in, :hidden].set(
        w1.astype(jnp.bfloat16))
    b1_p = jnp.zeros((1, h_pad), jnp.float32).at[0, :hidden].set(b1)
    w2_p = jnp.zeros((h_pad, h_pad), jnp.bfloat16).at[:hidden, :hidden].set(
        w2.astype(jnp.bfloat16))
    b2_p = jnp.zeros((1, h_pad), jnp.float32).at[0, :hidden].set(b2)

    a_p, dinv_p, p1 = pl.pallas_call(
        functools.partial(_build_kernel, tb=tb, lb=lb, n=n),
        grid_spec=pltpu.PrefetchScalarGridSpec(
            num_scalar_prefetch=1,
            grid=(nt, nt),
            in_specs=[
                pl.BlockSpec((1, e_pad + 128), lambda i, k, s: (0, 0)),
                pl.BlockSpec((tb, f_pad), lambda i, k, s: (i, 0)),
                pl.BlockSpec((f_pad, h_pad), lambda i, k, s: (0, 0)),
            ],
            out_specs=[
                pl.BlockSpec((tb, tb), lambda i, k, s: (i, k)),
                pl.BlockSpec((tb, 1), lambda i, k, s: (i, 0)),
                pl.BlockSpec((tb, h_pad), lambda i, k, s: (i, 0)),
            ],
            scratch_shapes=[pltpu.VMEM((tb, tb), jnp.int32),
                            pltpu.VMEM((tb, 1), jnp.int32)],
        ),
        out_shape=[jax.ShapeDtypeStruct((npad, npad), jnp.int8),
                   jax.ShapeDtypeStruct((npad, 1), jnp.float32),
                   jax.ShapeDtypeStruct((npad, h_pad), jnp.bfloat16)],
        compiler_params=pltpu.CompilerParams(
            dimension_semantics=("parallel", "arbitrary")),
    )(starts, key2d, x_p, w1_p)

    vmem_limit = min(
        int(2 * tm * tm
            + 2 * npad * h_pad * 2
            + 2 * tm * 128 * 4
            + 2 * 8 * h_pad * 4
            + h_pad * h_pad * 2
            + tm * h_pad * 4
            + 2 * tm * h_pad * 4) * 2
        + (8 << 20), 96 << 20)

    agg_cost = pl.CostEstimate(
        flops=2 * npad * npad * h_pad,
        transcendentals=0,
        bytes_accessed=npad * npad + npad * h_pad * 2 + npad * h_pad * 4
                       + npad * 4,
    )

    p2 = pl.pallas_call(
        _agg_proj_kernel,
        out_shape=jax.ShapeDtypeStruct((npad, h_pad), jnp.bfloat16),
        grid=(n_row, n_row),
        in_specs=[
            pl.BlockSpec((tm, tm), lambda i, k: (i, k)),
            pl.BlockSpec((npad, h_pad), lambda i, k: (0, 0)),
            pl.BlockSpec((tm, 1), lambda i, k: (i, 0)),
            pl.BlockSpec((1, h_pad), lambda i, k: (0, 0)),
            pl.BlockSpec((h_pad, h_pad), lambda i, k: (0, 0)),
        ],
        out_specs=pl.BlockSpec((tm, h_pad), lambda i, k: (i, 0)),
        scratch_shapes=[pltpu.VMEM((tm, h_pad), jnp.float32)],
        compiler_params=pltpu.CompilerParams(
            dimension_semantics=("parallel", "arbitrary"),
            vmem_limit_bytes=vmem_limit,
        ),
        cost_estimate=agg_cost,
    )(a_p, p1, dinv_p, b1_p, w2_p)

    out = pl.pallas_call(
        _agg_final_kernel,
        out_shape=jax.ShapeDtypeStruct((npad, h_pad), jnp.float32),
        grid=(n_row, n_row),
        in_specs=[
            pl.BlockSpec((tm, tm), lambda i, k: (i, k)),
            pl.BlockSpec((npad, h_pad), lambda i, k: (0, 0)),
            pl.BlockSpec((tm, 1), lambda i, k: (i, 0)),
            pl.BlockSpec((1, h_pad), lambda i, k: (0, 0)),
        ],
        out_specs=pl.BlockSpec((tm, h_pad), lambda i, k: (i, 0)),
        compiler_params=pltpu.CompilerParams(
            dimension_semantics=("parallel", "arbitrary"),
            vmem_limit_bytes=vmem_limit,
        ),
        cost_estimate=agg_cost,
    )(a_p, p2, dinv_p, b2_p)

    return out[:n, :hidden]
